# baseline (speedup 1.0000x reference)
import dataclasses
import functools

import jax
import jax.numpy as jnp
from jax import lax
from jax.experimental import pallas as pl
from jax.experimental.pallas import tpu as pltpu
from jax.experimental.pallas import tpu_sc as plsc

B, S, D = 1, 2048, 768
E, K = 16, 2
H = 3072
HC = 768
T = S * B
P = K * T
BLK = 256
NPAD = 8192
NB = NPAD // BLK

_VMESH = plsc.VectorSubcoreMesh(core_axis_name="c", subcore_axis_name="s")

_SC_PARAMS = pltpu.CompilerParams()
if "needs_layout_passes" in pltpu.CompilerParams.__dataclass_fields__:
    _SC_PARAMS = dataclasses.replace(_SC_PARAMS, needs_layout_passes=False)


def _gate_kernel(x_ref, wg_ref, e1_ref, e2_ref, p1_ref, p2_ref, aux_ref,
                 xb_ref):
    xb = x_ref[...].astype(jnp.bfloat16)
    xb_ref[...] = xb
    wgb = wg_ref[...].astype(jnp.bfloat16)
    logits = lax.dot_general(
        wgb, xb, (((1,), (1,)), ((), ())),
        preferred_element_type=jnp.float32)
    m = jnp.max(logits, axis=0, keepdims=True)
    ex = jnp.exp(logits - m)
    probs = ex / jnp.sum(ex, axis=0, keepdims=True)

    row = lax.broadcasted_iota(jnp.int32, probs.shape, 0)
    m1 = jnp.max(probs, axis=0, keepdims=True)
    e1 = jnp.min(jnp.where(probs == m1, row, E), axis=0, keepdims=True)
    probs2 = jnp.where(row == e1, -1.0, probs)
    m2 = jnp.max(probs2, axis=0, keepdims=True)
    e2 = jnp.min(jnp.where(probs2 == m2, row, E), axis=0, keepdims=True)

    denom = m1 + m2
    e1_ref[...] = e1
    e2_ref[...] = e2
    p1_ref[...] = m1 / denom
    p2_ref[...] = m2 / denom

    t = jnp.float32(probs.shape[1])
    mean_prob = jnp.sum(probs, axis=1, keepdims=True) / t
    ind = (probs > m2).astype(jnp.float32)
    mean_ind = jnp.sum(ind, axis=1, keepdims=True) / t
    aux_ref[...] = jnp.sum(mean_prob * mean_ind, keepdims=True).reshape(1, 1) * E


def _route_kernel(eall_hbm, pall_hbm, rt_hbm, pos_hbm, blk_hbm, rowp_hbm,
                  evv, rankv, posv, tokv, pvv, cntv, zv, zvf, bev, camat,
                  basev, shared_counts, dma_sem):
    cid = lax.axis_index("c")
    tid = lax.axis_index("s")
    CH = P // 16
    L = 16

    @pl.when(cid == 0)
    def _():
        base = tid * CH
        pltpu.sync_copy(eall_hbm.at[pl.ds(base, CH)], evv)
        pltpu.sync_copy(pall_hbm.at[pl.ds(base, 128)], pvv.at[0])
        pltpu.sync_copy(pall_hbm.at[pl.ds(base + 128, 128)], pvv.at[1])
        cntv[...] = jnp.zeros((E,), jnp.int32)

        @pl.loop(0, CH // L)
        def _(i):
            v = evv[pl.ds(i * L, L)]
            chunk_base = plsc.load_gather(cntv, [v])
            r = jnp.zeros((L,), jnp.int32)
            hist = jnp.zeros((E,), jnp.int32)
            lane = lax.iota(jnp.int32, L)
            for e in range(E):
                mi = (v == e).astype(jnp.int32)
                cs = plsc.cumsum(mi)
                r = jnp.where(v == e, cs - 1, r)
                hist = jnp.where(lane == e, jnp.sum(mi), hist)
            rankv[pl.ds(i * L, L)] = chunk_base + r
            cntv[...] = cntv[...] + hist

        pltpu.sync_copy(cntv, shared_counts.at[tid])
        @pl.loop(0, NPAD // 16 // 16)
        def _(i):
            zv[pl.ds(i * 16, 16)] = jnp.zeros((16,), jnp.int32)
            zvf[pl.ds(i * 16, 16)] = jnp.zeros((16,), jnp.float32)
        pltpu.sync_copy(zv, rt_hbm.at[pl.ds(tid * (NPAD // 16), NPAD // 16)])
        pltpu.sync_copy(
            zvf, rowp_hbm.at[pl.ds(tid * (NPAD // 16), NPAD // 16)])

        plsc.subcore_barrier()
        pltpu.sync_copy(shared_counts, camat)

        tot = jnp.zeros((E,), jnp.int32)
        pre = jnp.zeros((E,), jnp.int32)
        for k in range(16):
            row = camat[k, :]
            tot = tot + row
            pre = pre + jnp.where(jnp.full((E,), k) < tid, row, 0)
        padded = ((tot + (BLK - 1)) >> 8) << 8
        start = plsc.cumsum(padded) - padded
        basev[...] = start + pre

        @pl.loop(0, CH // L)
        def _(i):
            v = evv[pl.ds(i * L, L)]
            pos = plsc.load_gather(basev, [v]) + rankv[pl.ds(i * L, L)]
            j = i >> 3
            col = (i & 7) * L
            posv[j, pl.ds(col, L)] = pos
            tokv[j, pl.ds(col, L)] = (base + i * L + lax.iota(jnp.int32, L)) & (T - 1)

        pltpu.sync_copy(posv, pos_hbm.at[pl.ds(2 * tid, 2)])
        for j in range(2):
            pltpu.async_copy(tokv.at[j], rt_hbm.at[posv.at[j]], dma_sem).wait()
            pltpu.async_copy(pvv.at[j], rowp_hbm.at[posv.at[j]], dma_sem).wait()

        @pl.when(tid == 0)
        def _():
            end = start + padded
            for kv in range(NB // 16):
                bpos = (lax.iota(jnp.int32, L) + kv * L) * BLK
                ans = jnp.zeros((L,), jnp.int32)
                for e in range(E):
                    hit = jnp.logical_and(bpos >= start[e], bpos < end[e])
                    ans = jnp.where(hit, e, ans)
                bev[pl.ds(kv * L, L)] = ans
            pltpu.sync_copy(bev, blk_hbm)


def _dispatch_kernel(x_hbm, rt2d_hbm, xs_hbm, idxv, buf0, buf1, g0, g1, w0,
                     w1):
    w = lax.axis_index("s") * 2 + lax.axis_index("c")
    pltpu.sync_copy(rt2d_hbm.at[pl.ds(2 * w, 2)], idxv)
    cg0 = pltpu.async_copy(x_hbm.at[idxv.at[0]], buf0, g0)
    cg1 = pltpu.async_copy(x_hbm.at[idxv.at[1]], buf1, g1)
    cg0.wait()
    cw0 = pltpu.async_copy(buf0, xs_hbm.at[pl.ds(w * 256, 128)], w0)
    cg1.wait()
    cw1 = pltpu.async_copy(buf1, xs_hbm.at[pl.ds(w * 256 + 128, 128)], w1)
    cw0.wait()
    cw1.wait()


def _ffn_kernel(blk_ref, xs_ref, w1_ref, b1_ref, w2_ref, b2_ref, rowp_ref,
                ys_ref):
    xb = xs_ref[...]
    acc = jnp.zeros((BLK, D), jnp.float32)
    for hc in range(H // HC):
        w1c = w1_ref[0, hc * HC:(hc + 1) * HC, :]
        h = lax.dot_general(xb, w1c, (((1,), (1,)), ((), ())),
                            preferred_element_type=jnp.float32)
        h = h + b1_ref[0, 0, hc * HC:(hc + 1) * HC][None, :]
        h = jnp.maximum(h, 0.0).astype(jnp.bfloat16)
        w2c = w2_ref[0, :, hc * HC:(hc + 1) * HC]
        acc = acc + lax.dot_general(h, w2c, (((1,), (1,)), ((), ())),
                                    preferred_element_type=jnp.float32)
    ys_ref[...] = (acc + b2_ref[0, 0, :][None, :]) * rowp_ref[0, 0, :][:, None]


def _combine_kernel(ys_hbm, pos_hbm, out_hbm, idxv, r1, r2, ob, dma_sem):
    w = lax.axis_index("s") * 2 + lax.axis_index("c")
    for k in range(2):
        tb = w * 64 + k * 32
        pltpu.sync_copy(pos_hbm.at[pl.ds(tb, 32)], idxv.at[0])
        pltpu.sync_copy(pos_hbm.at[pl.ds(T + tb, 32)], idxv.at[1])
        pltpu.async_copy(ys_hbm.at[idxv.at[0]], r1, dma_sem).wait()
        pltpu.async_copy(ys_hbm.at[idxv.at[1]], r2, dma_sem).wait()

        @pl.loop(0, 32)
        def _(i):
            for c in range(D // 16):
                sl = pl.ds(c * 16, 16)
                ob[i, sl] = r1[i, sl] + r2[i, sl]

        pltpu.sync_copy(ob, out_hbm.at[pl.ds(tb, 32)])


def kernel(x, Wg, W1, b1, W2, b2):
    x_flat = x.reshape(T, D)

    e1, e2, p1, p2, aux, xb = pl.pallas_call(
        _gate_kernel,
        out_shape=(
            jax.ShapeDtypeStruct((1, T), jnp.int32),
            jax.ShapeDtypeStruct((1, T), jnp.int32),
            jax.ShapeDtypeStruct((1, T), jnp.float32),
            jax.ShapeDtypeStruct((1, T), jnp.float32),
            jax.ShapeDtypeStruct((1, 1), jnp.float32),
            jax.ShapeDtypeStruct((T, D), jnp.bfloat16),
        ),
    )(x_flat, Wg)

    eall = jnp.concatenate([e1.reshape(T), e2.reshape(T)])
    pall = jnp.concatenate([p1.reshape(T), p2.reshape(T)])

    cnt = jnp.zeros((E,), jnp.int32).at[eall].add(1)
    padded = (cnt + (BLK - 1)) // BLK * BLK
    startp = jnp.cumsum(padded) - padded
    startu = jnp.cumsum(cnt) - cnt
    order = jnp.argsort(eall, stable=True).astype(jnp.int32)
    se = eall[order]
    slot = startp[se] + (jnp.arange(P, dtype=jnp.int32) - startu[se])
    rt = jnp.zeros((NPAD,), jnp.int32).at[slot].set(order & (T - 1))
    rowp = jnp.zeros((NPAD,), jnp.float32).at[slot].set(pall[order])
    pos = jnp.zeros((P,), jnp.int32).at[order].set(slot)
    bpos = jnp.arange(NB, dtype=jnp.int32) * BLK
    ends = jnp.cumsum(padded)
    blk_e = jnp.minimum(
        jnp.searchsorted(ends, bpos, side="right"), E - 1).astype(jnp.int32)

    xb32 = lax.bitcast_convert_type(xb.reshape(T, D // 2, 2), jnp.int32)
    dispatch = pl.kernel(
        _dispatch_kernel,
        out_type=jax.ShapeDtypeStruct((NPAD, D // 2), jnp.int32),
        mesh=_VMESH,
        scratch_types=[
            pltpu.VMEM((2, 128), jnp.int32),
            pltpu.VMEM((128, D // 2), jnp.int32),
            pltpu.VMEM((128, D // 2), jnp.int32),
            pltpu.SemaphoreType.DMA,
            pltpu.SemaphoreType.DMA,
            pltpu.SemaphoreType.DMA,
            pltpu.SemaphoreType.DMA,
        ],
    )
    xs32 = dispatch(xb32, rt.reshape(NPAD // 128, 128))
    xs = lax.bitcast_convert_type(xs32, jnp.bfloat16).reshape(NPAD, D)

    w1_bf = W1.astype(jnp.bfloat16)
    w2_bf = W2.astype(jnp.bfloat16)
    grid_spec = pltpu.PrefetchScalarGridSpec(
        num_scalar_prefetch=1,
        grid=(NB,),
        in_specs=[
            pl.BlockSpec((BLK, D), lambda b, s: (b, 0)),
            pl.BlockSpec((1, H, D), lambda b, s: (s[b], 0, 0)),
            pl.BlockSpec((1, 1, H), lambda b, s: (s[b], 0, 0)),
            pl.BlockSpec((1, D, H), lambda b, s: (s[b], 0, 0)),
            pl.BlockSpec((1, 1, D), lambda b, s: (s[b], 0, 0)),
            pl.BlockSpec((1, 1, BLK), lambda b, s: (b, 0, 0)),
        ],
        out_specs=pl.BlockSpec((BLK, D), lambda b, s: (b, 0)),
    )
    ys = pl.pallas_call(
        _ffn_kernel,
        grid_spec=grid_spec,
        out_shape=jax.ShapeDtypeStruct((NPAD, D), jnp.float32),
    )(blk_e, xs, w1_bf, b1.reshape(E, 1, H), w2_bf, b2.reshape(E, 1, D),
      rowp.reshape(NB, 1, BLK))

    combine = pl.kernel(
        _combine_kernel,
        out_type=jax.ShapeDtypeStruct((T, D), jnp.float32),
        mesh=_VMESH,
        scratch_types=[
            pltpu.VMEM((2, 32), jnp.int32),
            pltpu.VMEM((32, D), jnp.float32),
            pltpu.VMEM((32, D), jnp.float32),
            pltpu.VMEM((32, D), jnp.float32),
            pltpu.SemaphoreType.DMA,
        ],
    )
    out = combine(ys, pos.reshape(P))

    return out.reshape(B, S, D), aux[0, 0]


# speedup vs baseline: 1.0986x; 1.0986x over previous
import dataclasses
import functools

import jax
import jax.numpy as jnp
from jax import lax
from jax.experimental import pallas as pl
from jax.experimental.pallas import tpu as pltpu
from jax.experimental.pallas import tpu_sc as plsc

B, S, D = 1, 2048, 768
E, K = 16, 2
H = 3072
HC = 768
T = S * B
P = K * T
BLK = 256
NPAD = 8192
NB = NPAD // BLK

_VMESH = plsc.VectorSubcoreMesh(core_axis_name="c", subcore_axis_name="s")

_SC_PARAMS = pltpu.CompilerParams()
if "needs_layout_passes" in pltpu.CompilerParams.__dataclass_fields__:
    _SC_PARAMS = dataclasses.replace(_SC_PARAMS, needs_layout_passes=False)


def _gate_kernel(x_ref, wg_ref, e1_ref, e2_ref, p1_ref, p2_ref, aux_ref,
                 xb_ref):
    xb = x_ref[...].astype(jnp.bfloat16)
    xb_ref[...] = xb
    wgb = wg_ref[...].astype(jnp.bfloat16)
    logits = lax.dot_general(
        wgb, xb, (((1,), (1,)), ((), ())),
        preferred_element_type=jnp.float32)
    m = jnp.max(logits, axis=0, keepdims=True)
    ex = jnp.exp(logits - m)
    probs = ex / jnp.sum(ex, axis=0, keepdims=True)

    row = lax.broadcasted_iota(jnp.int32, probs.shape, 0)
    m1 = jnp.max(probs, axis=0, keepdims=True)
    e1 = jnp.min(jnp.where(probs == m1, row, E), axis=0, keepdims=True)
    probs2 = jnp.where(row == e1, -1.0, probs)
    m2 = jnp.max(probs2, axis=0, keepdims=True)
    e2 = jnp.min(jnp.where(probs2 == m2, row, E), axis=0, keepdims=True)

    denom = m1 + m2
    e1_ref[...] = e1
    e2_ref[...] = e2
    p1_ref[...] = m1 / denom
    p2_ref[...] = m2 / denom

    t = jnp.float32(probs.shape[1])
    mean_prob = jnp.sum(probs, axis=1, keepdims=True) / t
    ind = (probs > m2).astype(jnp.float32)
    mean_ind = jnp.sum(ind, axis=1, keepdims=True) / t
    aux_ref[...] = jnp.sum(mean_prob * mean_ind, keepdims=True).reshape(1, 1) * E


def _route_kernel(eall_hbm, pall_hbm, rt_hbm, pos_hbm, blk_hbm, rowp_hbm,
                  evv, rankv, posv, tokv, pvv, cntv, zv, zvf, bev, camat,
                  basev, shared_counts, dma_sem):
    cid = lax.axis_index("c")
    tid = lax.axis_index("s")
    CH = P // 16
    L = 16

    @pl.when(cid == 0)
    def _():
        base = tid * CH
        pltpu.sync_copy(eall_hbm.at[pl.ds(base, CH)], evv)
        pltpu.sync_copy(pall_hbm.at[pl.ds(base, 128)], pvv.at[0])
        pltpu.sync_copy(pall_hbm.at[pl.ds(base + 128, 128)], pvv.at[1])
        cntv[...] = jnp.zeros((E,), jnp.int32)

        @pl.loop(0, CH // L)
        def _(i):
            v = evv[pl.ds(i * L, L)]
            chunk_base = plsc.load_gather(cntv, [v])
            r = jnp.zeros((L,), jnp.int32)
            hist = jnp.zeros((E,), jnp.int32)
            lane = lax.iota(jnp.int32, L)
            for e in range(E):
                mi = (v == e).astype(jnp.int32)
                cs = plsc.cumsum(mi)
                r = jnp.where(v == e, cs - 1, r)
                hist = jnp.where(lane == e, jnp.sum(mi), hist)
            rankv[pl.ds(i * L, L)] = chunk_base + r
            cntv[...] = cntv[...] + hist

        pltpu.sync_copy(cntv, shared_counts.at[tid])
        @pl.loop(0, NPAD // 16 // 16)
        def _(i):
            zv[pl.ds(i * 16, 16)] = jnp.zeros((16,), jnp.int32)
            zvf[pl.ds(i * 16, 16)] = jnp.zeros((16,), jnp.float32)
        pltpu.sync_copy(zv, rt_hbm.at[pl.ds(tid * (NPAD // 16), NPAD // 16)])
        pltpu.sync_copy(
            zvf, rowp_hbm.at[pl.ds(tid * (NPAD // 16), NPAD // 16)])

        plsc.subcore_barrier()
        pltpu.sync_copy(shared_counts, camat)

        tot = jnp.zeros((E,), jnp.int32)
        pre = jnp.zeros((E,), jnp.int32)
        for k in range(16):
            row = camat[k, :]
            tot = tot + row
            pre = pre + jnp.where(jnp.full((E,), k) < tid, row, 0)
        padded = ((tot + (BLK - 1)) >> 8) << 8
        start = plsc.cumsum(padded) - padded
        basev[...] = start + pre

        @pl.loop(0, CH // L)
        def _(i):
            v = evv[pl.ds(i * L, L)]
            pos = plsc.load_gather(basev, [v]) + rankv[pl.ds(i * L, L)]
            j = i >> 3
            col = (i & 7) * L
            posv[j, pl.ds(col, L)] = pos
            tokv[j, pl.ds(col, L)] = (base + i * L + lax.iota(jnp.int32, L)) & (T - 1)

        pltpu.sync_copy(posv, pos_hbm.at[pl.ds(2 * tid, 2)])
        for j in range(2):
            pltpu.async_copy(tokv.at[j], rt_hbm.at[posv.at[j]], dma_sem).wait()
            pltpu.async_copy(pvv.at[j], rowp_hbm.at[posv.at[j]], dma_sem).wait()

        @pl.when(tid == 0)
        def _():
            end = start + padded
            for kv in range(NB // 16):
                bpos = (lax.iota(jnp.int32, L) + kv * L) * BLK
                ans = jnp.zeros((L,), jnp.int32)
                for e in range(E):
                    hit = jnp.logical_and(bpos >= start[e], bpos < end[e])
                    ans = jnp.where(hit, e, ans)
                bev[pl.ds(kv * L, L)] = ans
            pltpu.sync_copy(bev, blk_hbm)


def _dispatch_kernel(x_hbm, rt2d_hbm, xs_hbm, idxv, buf, dma_sem):
    w = lax.axis_index("s") * 2 + lax.axis_index("c")
    pltpu.sync_copy(rt2d_hbm.at[pl.ds(w, 1)], idxv)
    pltpu.async_copy(x_hbm.at[idxv.at[0]], buf, dma_sem).wait()
    pltpu.sync_copy(buf, xs_hbm.at[pl.ds(w * 128, 128)])


def _ffn_kernel(blk_ref, xs_ref, w1_ref, b1_ref, w2_ref, b2_ref, rowp_ref,
                ys_ref):
    xb = xs_ref[...].astype(jnp.bfloat16)
    acc = jnp.zeros((BLK, D), jnp.float32)
    for hc in range(H // HC):
        w1c = w1_ref[0, hc * HC:(hc + 1) * HC, :]
        h = lax.dot_general(xb, w1c, (((1,), (1,)), ((), ())),
                            preferred_element_type=jnp.float32)
        h = h + b1_ref[0, 0, hc * HC:(hc + 1) * HC][None, :]
        h = jnp.maximum(h, 0.0).astype(jnp.bfloat16)
        w2c = w2_ref[0, :, hc * HC:(hc + 1) * HC]
        acc = acc + lax.dot_general(h, w2c, (((1,), (1,)), ((), ())),
                                    preferred_element_type=jnp.float32)
    ys_ref[...] = (acc + b2_ref[0, 0, :][None, :]) * rowp_ref[0, 0, :][:, None]


def _combine_kernel(ys_hbm, pos_hbm, out_hbm, idxv, r1, r2, ob, dma_sem):
    w = lax.axis_index("s") * 2 + lax.axis_index("c")
    for k in range(2):
        tb = w * 64 + k * 32
        pltpu.sync_copy(pos_hbm.at[pl.ds(tb, 32)], idxv.at[0])
        pltpu.sync_copy(pos_hbm.at[pl.ds(T + tb, 32)], idxv.at[1])
        pltpu.async_copy(ys_hbm.at[idxv.at[0]], r1, dma_sem).wait()
        pltpu.async_copy(ys_hbm.at[idxv.at[1]], r2, dma_sem).wait()

        @pl.loop(0, 32)
        def _(i):
            for c in range(D // 16):
                sl = pl.ds(c * 16, 16)
                ob[i, sl] = r1[i, sl] + r2[i, sl]

        pltpu.sync_copy(ob, out_hbm.at[pl.ds(tb, 32)])


def kernel(x, Wg, W1, b1, W2, b2):
    x_flat = x.reshape(T, D)

    e1, e2, p1, p2, aux, xb = pl.pallas_call(
        _gate_kernel,
        out_shape=(
            jax.ShapeDtypeStruct((1, T), jnp.int32),
            jax.ShapeDtypeStruct((1, T), jnp.int32),
            jax.ShapeDtypeStruct((1, T), jnp.float32),
            jax.ShapeDtypeStruct((1, T), jnp.float32),
            jax.ShapeDtypeStruct((1, 1), jnp.float32),
            jax.ShapeDtypeStruct((T, D), jnp.bfloat16),
        ),
    )(x_flat, Wg)

    eall = jnp.concatenate([e1.reshape(T), e2.reshape(T)])
    pall = jnp.concatenate([p1.reshape(T), p2.reshape(T)])

    cnt = jnp.zeros((E,), jnp.int32).at[eall].add(1)
    padded = (cnt + (BLK - 1)) // BLK * BLK
    startp = jnp.cumsum(padded) - padded
    startu = jnp.cumsum(cnt) - cnt
    order = jnp.argsort(eall, stable=True).astype(jnp.int32)
    se = eall[order]
    slot = startp[se] + (jnp.arange(P, dtype=jnp.int32) - startu[se])
    rt = jnp.zeros((NPAD,), jnp.int32).at[slot].set(order & (T - 1))
    rowp = jnp.zeros((NPAD,), jnp.float32).at[slot].set(pall[order])
    pos = jnp.zeros((P,), jnp.int32).at[order].set(slot)
    bpos = jnp.arange(NB, dtype=jnp.int32) * BLK
    ends = jnp.cumsum(padded)
    blk_e = jnp.minimum(
        jnp.searchsorted(ends, bpos, side="right"), E - 1).astype(jnp.int32)

    w1_bf = W1.astype(jnp.bfloat16)
    w2_bf = W2.astype(jnp.bfloat16)
    NH = NPAD // 2
    NBH = NB // 2

    dispatch = pl.kernel(
        _dispatch_kernel,
        out_type=jax.ShapeDtypeStruct((NH, D), jnp.float32),
        mesh=_VMESH,
        scratch_types=[
            pltpu.VMEM((1, 128), jnp.int32),
            pltpu.VMEM((128, D), jnp.float32),
            pltpu.SemaphoreType.DMA,
        ],
    )

    grid_spec = pltpu.PrefetchScalarGridSpec(
        num_scalar_prefetch=1,
        grid=(NBH,),
        in_specs=[
            pl.BlockSpec((BLK, D), lambda b, s: (b, 0)),
            pl.BlockSpec((1, H, D), lambda b, s: (s[b], 0, 0)),
            pl.BlockSpec((1, 1, H), lambda b, s: (s[b], 0, 0)),
            pl.BlockSpec((1, D, H), lambda b, s: (s[b], 0, 0)),
            pl.BlockSpec((1, 1, D), lambda b, s: (s[b], 0, 0)),
            pl.BlockSpec((1, 1, BLK), lambda b, s: (b, 0, 0)),
        ],
        out_specs=pl.BlockSpec((BLK, D), lambda b, s: (b, 0)),
    )
    rt2d = rt.reshape(NPAD // 128, 128)
    b1r = b1.reshape(E, 1, H)
    b2r = b2.reshape(E, 1, D)
    rowpr = rowp.reshape(NB, 1, BLK)

    ys_halves = []
    for h in range(2):
        xs_h = dispatch(x_flat, rt2d[h * (NH // 128):(h + 1) * (NH // 128)])
        ys_h = pl.pallas_call(
            _ffn_kernel,
            grid_spec=grid_spec,
            out_shape=jax.ShapeDtypeStruct((NH, D), jnp.float32),
        )(blk_e[h * NBH:(h + 1) * NBH], xs_h, w1_bf, b1r, w2_bf, b2r,
          rowpr[h * NBH:(h + 1) * NBH])
        ys_halves.append(ys_h)
    ys = jnp.concatenate(ys_halves)

    combine = pl.kernel(
        _combine_kernel,
        out_type=jax.ShapeDtypeStruct((T, D), jnp.float32),
        mesh=_VMESH,
        scratch_types=[
            pltpu.VMEM((2, 32), jnp.int32),
            pltpu.VMEM((32, D), jnp.float32),
            pltpu.VMEM((32, D), jnp.float32),
            pltpu.VMEM((32, D), jnp.float32),
            pltpu.SemaphoreType.DMA,
        ],
    )
    out = combine(ys, pos.reshape(P))

    return out.reshape(B, S, D), aux[0, 0]


# speedup vs baseline: 1.3024x; 1.1855x over previous
import functools

import jax
import jax.numpy as jnp
from jax.experimental import pallas as pl

B, S, D = 1, 2048, 768
E, K = 16, 2
H = 3072
HC = 768


def _gate_kernel(x_ref, wg_ref, comb_ref, aux_ref):
    x = x_ref[...].astype(jnp.bfloat16)
    wg = wg_ref[...].astype(jnp.bfloat16)
    logits = jax.lax.dot_general(
        x, wg, (((1,), (1,)), ((), ())),
        preferred_element_type=jnp.float32,
    )
    m = jnp.max(logits, axis=-1, keepdims=True)
    ex = jnp.exp(logits - m)
    probs = ex / jnp.sum(ex, axis=-1, keepdims=True)

    lane = jax.lax.broadcasted_iota(jnp.int32, probs.shape, 1)
    m1 = jnp.max(probs, axis=-1, keepdims=True)
    i1 = jnp.min(jnp.where(probs == m1, lane, E), axis=-1, keepdims=True)
    probs2 = jnp.where(lane == i1, -1.0, probs)
    m2 = jnp.max(probs2, axis=-1, keepdims=True)
    i2 = jnp.min(jnp.where(probs2 == m2, lane, E), axis=-1, keepdims=True)

    denom = m1 + m2
    p1 = m1 / denom
    p2 = m2 / denom
    comb = jnp.where(lane == i1, p1, 0.0) + jnp.where(lane == i2, p2, 0.0)
    comb_ref[...] = comb

    t = jnp.float32(probs.shape[0])
    mean_prob = jnp.sum(probs, axis=0, keepdims=True) / t
    ind = (probs > m2).astype(jnp.float32)
    mean_ind = jnp.sum(ind, axis=0, keepdims=True) / t
    aux_ref[...] = jnp.sum(mean_prob * mean_ind, keepdims=True).reshape(1, 1) * E


def _expert_kernel(x_ref, w1_ref, b1_ref, w2_ref, b2_ref, comb_ref, out_ref):
    e = pl.program_id(0)
    x = x_ref[...]
    lane = jax.lax.broadcasted_iota(jnp.int32, comb_ref.shape, 1)
    c = jnp.sum(jnp.where(lane == e, comb_ref[...], 0.0), axis=-1, keepdims=True)

    acc = jnp.zeros((x.shape[0], D), jnp.float32)
    for hc in range(H // HC):
        w1c = w1_ref[0, hc * HC:(hc + 1) * HC, :]
        h = jax.lax.dot_general(
            x, w1c, (((1,), (1,)), ((), ())),
            preferred_element_type=jnp.float32)
        h = h + b1_ref[0, 0, hc * HC:(hc + 1) * HC][None, :]
        h = jnp.maximum(h, 0.0).astype(jnp.bfloat16)
        w2c = w2_ref[0, :, hc * HC:(hc + 1) * HC]
        acc = acc + jax.lax.dot_general(
            h, w2c, (((1,), (1,)), ((), ())),
            preferred_element_type=jnp.float32)
    acc = acc + b2_ref[0, 0, :][None, :]

    @pl.when(e == 0)
    def _():
        out_ref[...] = jnp.zeros_like(out_ref)

    out_ref[...] += c * acc


def kernel(x, Wg, W1, b1, W2, b2):
    x_flat = x.reshape(-1, D)
    T = x_flat.shape[0]

    comb, aux = pl.pallas_call(
        _gate_kernel,
        out_shape=(
            jax.ShapeDtypeStruct((T, E), jnp.float32),
            jax.ShapeDtypeStruct((1, 1), jnp.float32),
        ),
    )(x_flat, Wg)

    x_bf = x_flat.astype(jnp.bfloat16)
    w1_bf = W1.astype(jnp.bfloat16)
    w2_bf = W2.astype(jnp.bfloat16)

    out = pl.pallas_call(
        _expert_kernel,
        grid=(E,),
        in_specs=[
            pl.BlockSpec((T, D), lambda e: (0, 0)),
            pl.BlockSpec((1, H, D), lambda e: (e, 0, 0)),
            pl.BlockSpec((1, 1, H), lambda e: (e, 0, 0)),
            pl.BlockSpec((1, D, H), lambda e: (e, 0, 0)),
            pl.BlockSpec((1, 1, D), lambda e: (e, 0, 0)),
            pl.BlockSpec((T, E), lambda e: (0, 0)),
        ],
        out_specs=pl.BlockSpec((T, D), lambda e: (0, 0)),
        out_shape=jax.ShapeDtypeStruct((T, D), jnp.float32),
    )(x_bf, w1_bf, b1.reshape(E, 1, H), w2_bf, b2.reshape(E, 1, D), comb)

    return out.reshape(B, S, D), aux[0, 0]
